# Initial kernel scaffold; baseline (speedup 1.0000x reference)
import jax
import jax.numpy as jnp
from jax.experimental import pallas as pl


def kernel(feature_maps, landmarks):
    raise NotImplementedError("write your pallas kernel here")


# speedup vs baseline: 5.9558x; 5.9558x over previous
import functools

import jax
import jax.numpy as jnp
from jax.experimental import pallas as pl
from jax.experimental.pallas import tpu as pltpu

_RADIUS = 40
_RAD2 = _RADIUS * _RADIUS


def _loss_body(P, h, w, xy_ref, l_ref, ox_ref, oy_ref, out_ref):
    g = pl.program_id(0)
    x = xy_ref[2 * g]
    y = xy_ref[2 * g + 1]
    r = jax.lax.broadcasted_iota(jnp.int32, (h, w), 0)
    c = jax.lax.broadcasted_iota(jnp.int32, (h, w), 1)
    dr = r - x
    dc = c - y
    mask = ((dr * dr + dc * dc) <= _RAD2).astype(jnp.float32)

    l = l_ref[0, 0]
    bce_sum = jnp.sum(
        jnp.maximum(l, 0.0) + jnp.log1p(jnp.exp(-jnp.abs(l))) - l * mask
    )
    cnt = jnp.sum(mask)
    omx = dr.astype(jnp.float32) * (-1.0 / _RADIUS)
    omy = dc.astype(jnp.float32) * (-1.0 / _RADIUS)
    l1x = jnp.sum(jnp.abs(ox_ref[0, 0] - omx) * mask)
    l1y = jnp.sum(jnp.abs(oy_ref[0, 0] - omy) * mask)
    val = 2.0 * bce_sum / (h * w) + (l1x + l1y) / cnt
    out_ref[0, 0, :] = jnp.full((128,), val, dtype=jnp.float32)


def kernel(feature_maps, landmarks):
    B, F, h, w = feature_maps.shape
    P = F // 3
    G = B * P
    lm = (landmarks * jnp.array([h, w], dtype=jnp.float32)).astype(jnp.int32)
    xy = lm.reshape(-1)

    grid_spec = pltpu.PrefetchScalarGridSpec(
        num_scalar_prefetch=1,
        grid=(G,),
        in_specs=[
            pl.BlockSpec((1, 1, h, w), lambda g, xy: (g // P, g % P, 0, 0)),
            pl.BlockSpec((1, 1, h, w), lambda g, xy: (g // P, P + g % P, 0, 0)),
            pl.BlockSpec((1, 1, h, w), lambda g, xy: (g // P, 2 * P + g % P, 0, 0)),
        ],
        out_specs=pl.BlockSpec((1, 1, 128), lambda g, xy: (g, 0, 0)),
    )
    per_pair = pl.pallas_call(
        functools.partial(_loss_body, P, h, w),
        grid_spec=grid_spec,
        out_shape=jax.ShapeDtypeStruct((G, 1, 128), jnp.float32),
    )(xy, feature_maps, feature_maps, feature_maps)
    return jnp.mean(per_pair[:, 0, 0])


# speedup vs baseline: 6.3015x; 1.0580x over previous
import functools

import jax
import jax.numpy as jnp
from jax.experimental import pallas as pl
from jax.experimental.pallas import tpu as pltpu

_RADIUS = 40
_RAD2 = _RADIUS * _RADIUS
_BLK = 128


def _win_base(v):
    lo = jnp.maximum(v - _RADIUS, 0)
    return jnp.minimum(lo // _BLK, 512 // _BLK - 2)


def _loss_body(P, h, w, xy_ref, l_ref,
               ox00, ox01, ox10, ox11, oy00, oy01, oy10, oy11, out_ref):
    g = pl.program_id(0)
    x = xy_ref[2 * g]
    y = xy_ref[2 * g + 1]

    r = jax.lax.broadcasted_iota(jnp.int32, (h, w), 0)
    c = jax.lax.broadcasted_iota(jnp.int32, (h, w), 1)
    dr = r - x
    dc = c - y
    mask = ((dr * dr + dc * dc) <= _RAD2).astype(jnp.float32)
    l = l_ref[0, 0]
    bce_sum = jnp.sum(
        jnp.maximum(l, 0.0) + jnp.log1p(jnp.exp(-jnp.abs(l))) - l * mask
    )
    cnt = jnp.sum(mask)

    rb = _win_base(x)
    cb = _win_base(y)
    l1 = jnp.float32(0.0)
    for i, j, is_x, oref in (
            (0, 0, True, ox00), (0, 1, True, ox01),
            (1, 0, True, ox10), (1, 1, True, ox11),
            (0, 0, False, oy00), (0, 1, False, oy01),
            (1, 0, False, oy10), (1, 1, False, oy11)):
        r0 = (rb + i) * _BLK
        c0 = (cb + j) * _BLK
        wr = r0 + jax.lax.broadcasted_iota(jnp.int32, (_BLK, _BLK), 0) - x
        wc = c0 + jax.lax.broadcasted_iota(jnp.int32, (_BLK, _BLK), 1) - y
        wm = ((wr * wr + wc * wc) <= _RAD2).astype(jnp.float32)
        ramp = (wr if is_x else wc).astype(jnp.float32) * (-1.0 / _RADIUS)
        l1 = l1 + jnp.sum(jnp.abs(oref[0, 0] - ramp) * wm)
    val = 2.0 * bce_sum / (h * w) + l1 / cnt
    out_ref[0, 0, :] = jnp.full((128,), val, dtype=jnp.float32)


def kernel(feature_maps, landmarks):
    B, F, h, w = feature_maps.shape
    P = F // 3
    G = B * P
    lm = (landmarks * jnp.array([h, w], dtype=jnp.float32)).astype(jnp.int32)
    xy = lm.reshape(-1)

    def win_spec(ch_off, i, j):
        def imap(g, xy):
            x = xy[2 * g]
            y = xy[2 * g + 1]
            return (g // P, ch_off + g % P, _win_base(x) + i, _win_base(y) + j)
        return pl.BlockSpec((1, 1, _BLK, _BLK), imap)

    in_specs = [pl.BlockSpec((1, 1, h, w), lambda g, xy: (g // P, g % P, 0, 0))]
    for ch in (P, 2 * P):
        for i in (0, 1):
            for j in (0, 1):
                in_specs.append(win_spec(ch, i, j))

    grid_spec = pltpu.PrefetchScalarGridSpec(
        num_scalar_prefetch=1,
        grid=(G,),
        in_specs=in_specs,
        out_specs=pl.BlockSpec((1, 1, 128), lambda g, xy: (g, 0, 0)),
    )
    per_pair = pl.pallas_call(
        functools.partial(_loss_body, P, h, w),
        grid_spec=grid_spec,
        out_shape=jax.ShapeDtypeStruct((G, 1, 128), jnp.float32),
    )(xy, *([feature_maps] * 9))
    return jnp.mean(per_pair[:, 0, 0])


# speedup vs baseline: 6.6260x; 1.0515x over previous
import functools

import jax
import jax.numpy as jnp
from jax.experimental import pallas as pl
from jax.experimental.pallas import tpu as pltpu

_RADIUS = 40
_RAD2 = _RADIUS * _RADIUS
_BLK = 128


def _win_base(v):
    lo = jnp.maximum(v - _RADIUS, 0)
    return jnp.minimum(lo // _BLK, 512 // _BLK - 2)


def _loss_body(P, h, w, xy_ref, l_ref,
               lw00, lw01, lw10, lw11,
               ox00, ox01, ox10, ox11,
               oy00, oy01, oy10, oy11, out_ref):
    g = pl.program_id(0)
    x = xy_ref[2 * g]
    y = xy_ref[2 * g + 1]

    l = l_ref[0, 0]
    bce_sum = jnp.sum(jnp.maximum(l, 0.0) + jnp.log1p(jnp.exp(-jnp.abs(l))))

    rb = _win_base(x)
    cb = _win_base(y)
    lwin = ((0, 0, lw00), (0, 1, lw01), (1, 0, lw10), (1, 1, lw11))
    oxwin = (ox00, ox01, ox10, ox11)
    oywin = (oy00, oy01, oy10, oy11)
    disk_l = jnp.float32(0.0)
    cnt = jnp.float32(0.0)
    l1 = jnp.float32(0.0)
    for k, (i, j, lref) in enumerate(lwin):
        r0 = (rb + i) * _BLK
        c0 = (cb + j) * _BLK
        wr = r0 + jax.lax.broadcasted_iota(jnp.int32, (_BLK, _BLK), 0) - x
        wc = c0 + jax.lax.broadcasted_iota(jnp.int32, (_BLK, _BLK), 1) - y
        wm = ((wr * wr + wc * wc) <= _RAD2).astype(jnp.float32)
        disk_l = disk_l + jnp.sum(lref[0, 0] * wm)
        cnt = cnt + jnp.sum(wm)
        rampx = wr.astype(jnp.float32) * (-1.0 / _RADIUS)
        rampy = wc.astype(jnp.float32) * (-1.0 / _RADIUS)
        l1 = l1 + jnp.sum((jnp.abs(oxwin[k][0, 0] - rampx)
                           + jnp.abs(oywin[k][0, 0] - rampy)) * wm)
    val = 2.0 * (bce_sum - disk_l) / (h * w) + l1 / cnt
    out_ref[0, 0, :] = jnp.full((128,), val, dtype=jnp.float32)


def kernel(feature_maps, landmarks):
    B, F, h, w = feature_maps.shape
    P = F // 3
    G = B * P
    lm = (landmarks * jnp.array([h, w], dtype=jnp.float32)).astype(jnp.int32)
    xy = lm.reshape(-1)

    def win_spec(ch_off, i, j):
        def imap(g, xy):
            x = xy[2 * g]
            y = xy[2 * g + 1]
            return (g // P, ch_off + g % P, _win_base(x) + i, _win_base(y) + j)
        return pl.BlockSpec((1, 1, _BLK, _BLK), imap)

    in_specs = [pl.BlockSpec((1, 1, h, w), lambda g, xy: (g // P, g % P, 0, 0))]
    for ch in (0, P, 2 * P):
        for i in (0, 1):
            for j in (0, 1):
                in_specs.append(win_spec(ch, i, j))

    grid_spec = pltpu.PrefetchScalarGridSpec(
        num_scalar_prefetch=1,
        grid=(G,),
        in_specs=in_specs,
        out_specs=pl.BlockSpec((1, 1, 128), lambda g, xy: (g, 0, 0)),
    )
    per_pair = pl.pallas_call(
        functools.partial(_loss_body, P, h, w),
        grid_spec=grid_spec,
        out_shape=jax.ShapeDtypeStruct((G, 1, 128), jnp.float32),
        compiler_params=pltpu.CompilerParams(
            dimension_semantics=("parallel",)),
    )(xy, *([feature_maps] * 13))
    return jnp.mean(per_pair[:, 0, 0])


# speedup vs baseline: 8.3282x; 1.2569x over previous
import functools

import jax
import jax.numpy as jnp
from jax.experimental import pallas as pl
from jax.experimental.pallas import tpu as pltpu

_RADIUS = 40
_RAD2 = _RADIUS * _RADIUS
_BLK = 128


def _win_base(v):
    lo = jnp.maximum(v - _RADIUS, 0)
    return jnp.minimum(lo // _BLK, 512 // _BLK - 2)


def _loss_body(P, h, w, xy_ref, l_ref,
               lw00, lw01, lw10, lw11,
               ox00, ox01, ox10, ox11,
               oy00, oy01, oy10, oy11, out_ref):
    g = pl.program_id(0)
    x = xy_ref[2 * g]
    y = xy_ref[2 * g + 1]

    _CH = 32
    acc = jnp.zeros((_CH, w), jnp.float32)
    for i in range(h // _CH):
        chunk = l_ref[0, 0, i * _CH:(i + 1) * _CH, :]
        acc = acc + (jnp.maximum(chunk, 0.0)
                     + jnp.log(1.0 + jnp.exp(-jnp.abs(chunk))))
    bce_sum = jnp.sum(acc)

    rb = _win_base(x)
    cb = _win_base(y)
    lwin = ((0, 0, lw00), (0, 1, lw01), (1, 0, lw10), (1, 1, lw11))
    oxwin = (ox00, ox01, ox10, ox11)
    oywin = (oy00, oy01, oy10, oy11)
    disk_l = jnp.float32(0.0)
    cnt = jnp.float32(0.0)
    l1 = jnp.float32(0.0)
    for k, (i, j, lref) in enumerate(lwin):
        r0 = (rb + i) * _BLK
        c0 = (cb + j) * _BLK
        wr = r0 + jax.lax.broadcasted_iota(jnp.int32, (_BLK, _BLK), 0) - x
        wc = c0 + jax.lax.broadcasted_iota(jnp.int32, (_BLK, _BLK), 1) - y
        wm = ((wr * wr + wc * wc) <= _RAD2).astype(jnp.float32)
        disk_l = disk_l + jnp.sum(lref[0, 0] * wm)
        cnt = cnt + jnp.sum(wm)
        rampx = wr.astype(jnp.float32) * (-1.0 / _RADIUS)
        rampy = wc.astype(jnp.float32) * (-1.0 / _RADIUS)
        l1 = l1 + jnp.sum((jnp.abs(oxwin[k][0, 0] - rampx)
                           + jnp.abs(oywin[k][0, 0] - rampy)) * wm)
    val = 2.0 * (bce_sum - disk_l) / (h * w) + l1 / cnt
    out_ref[0, 0, :] = jnp.full((128,), val, dtype=jnp.float32)


def kernel(feature_maps, landmarks):
    B, F, h, w = feature_maps.shape
    P = F // 3
    G = B * P
    lm = (landmarks * jnp.array([h, w], dtype=jnp.float32)).astype(jnp.int32)
    xy = lm.reshape(-1)

    def win_spec(ch_off, i, j):
        def imap(g, xy):
            x = xy[2 * g]
            y = xy[2 * g + 1]
            return (g // P, ch_off + g % P, _win_base(x) + i, _win_base(y) + j)
        return pl.BlockSpec((1, 1, _BLK, _BLK), imap)

    in_specs = [pl.BlockSpec((1, 1, h, w), lambda g, xy: (g // P, g % P, 0, 0))]
    for ch in (0, P, 2 * P):
        for i in (0, 1):
            for j in (0, 1):
                in_specs.append(win_spec(ch, i, j))

    grid_spec = pltpu.PrefetchScalarGridSpec(
        num_scalar_prefetch=1,
        grid=(G,),
        in_specs=in_specs,
        out_specs=pl.BlockSpec((1, 1, 128), lambda g, xy: (g, 0, 0)),
    )
    per_pair = pl.pallas_call(
        functools.partial(_loss_body, P, h, w),
        grid_spec=grid_spec,
        out_shape=jax.ShapeDtypeStruct((G, 1, 128), jnp.float32),
        compiler_params=pltpu.CompilerParams(
            dimension_semantics=("parallel",)),
    )(xy, *([feature_maps] * 13))
    return jnp.mean(per_pair[:, 0, 0])


# speedup vs baseline: 10.0711x; 1.2093x over previous
import functools

import jax
import jax.numpy as jnp
from jax import lax
from jax.experimental import pallas as pl
from jax.experimental.pallas import tpu as pltpu
from jax.experimental.pallas import tpu_sc as plsc

_RADIUS = 40
_RAD2 = _RADIUS * _RADIUS
_ROWS = 88
_SLABC = 256
_COLS = 96
_LANES = 16


def _dense_body(P, l_ref, out_ref, acc_ref):
    @pl.when((pl.program_id(0) == 0) & (pl.program_id(1) == 0))
    def _():
        acc_ref[...] = jnp.zeros(acc_ref.shape, acc_ref.dtype)

    acc = acc_ref[...]
    for ch in range(P):
        t = l_ref[0, ch]
        e = jnp.exp2(jnp.abs(t) * (-1.4426950408889634))
        acc = acc + (jnp.maximum(t, 0.0)
                     + jnp.log2(1.0 + e) * 0.6931471805599453)
    acc_ref[...] = acc

    @pl.when((pl.program_id(0) == pl.num_programs(0) - 1)
             & (pl.program_id(1) == pl.num_programs(1) - 1))
    def _():
        out_ref[...] = acc_ref[...]


def _dense_softplus_sum(feature_maps):
    B, F, h, w = feature_maps.shape
    P = F // 3
    out = pl.pallas_call(
        functools.partial(_dense_body, P),
        grid=(B, h // 32),
        in_specs=[pl.BlockSpec((1, P, 32, w), lambda b, r: (b, 0, r, 0))],
        out_specs=pl.BlockSpec((32, w), lambda b, r: (0, 0)),
        out_shape=jax.ShapeDtypeStruct((32, w), jnp.float32),
        scratch_shapes=[pltpu.VMEM((32, w), jnp.float32)],
    )(feature_maps)
    return jnp.sum(out)


def _window_body(B, P, h, w, fm, xy_hbm, o_hbm,
                 slab_l, slab_x, slab_y, accv, xy_smem,
                 sem_l, sem_x, sem_y, sem_o):
    G = B * P
    cid = lax.axis_index("c")
    sid = lax.axis_index("s")
    idx = cid * 16 + sid
    pltpu.async_copy(xy_hbm, xy_smem, sem_l).wait()
    lane = lax.broadcasted_iota(jnp.int32, (_LANES,), 0)

    for k in range((G + 31) // 32):
        g = idx + 32 * k

        @pl.when(g < G)
        def _():
            b = g // P
            p = g % P
            xyv = xy_smem[pl.ds(2 * g, _LANES)]
            x = xyv[0]
            y = xyv[1]
            rs = jnp.minimum((jnp.maximum(x - _RADIUS, 0) // 8) * 8, h - _ROWS)
            cs = jnp.minimum((jnp.maximum(y - _RADIUS, 0) // 128) * 128,
                             w - _SLABC)
            rs = pl.multiple_of(rs, 8)
            cs = pl.multiple_of(cs, 128)
            js = jnp.minimum((jnp.maximum(y - _RADIUS, 0) - cs) // _LANES,
                             (_SLABC - _COLS) // _LANES) * _LANES
            cp_l = pltpu.async_copy(
                fm.at[b, p, pl.ds(rs, _ROWS), pl.ds(cs, _SLABC)], slab_l, sem_l)
            cp_x = pltpu.async_copy(
                fm.at[b, P + p, pl.ds(rs, _ROWS), pl.ds(cs, _SLABC)],
                slab_x, sem_x)
            cp_y = pltpu.async_copy(
                fm.at[b, 2 * P + p, pl.ds(rs, _ROWS), pl.ds(cs, _SLABC)],
                slab_y, sem_y)
            cp_l.wait()
            cp_x.wait()
            cp_y.wait()

            zero = jnp.zeros((_LANES,), jnp.float32)

            def row_body(i, accs):
                d, n, ax, ay = accs
                r = rs + i
                dr = r - x
                dr2 = dr * dr
                rampx = (x - r).astype(jnp.float32) * (1.0 / _RADIUS)
                for j in range(_COLS // _LANES):
                    dc = cs + js + (16 * j) + lane - y
                    m = (dc * dc + dr2) <= _RAD2
                    lrow = slab_l[i, pl.ds(js + 16 * j, 16)]
                    d = d + jnp.where(m, lrow, 0.0)
                    n = n + jnp.where(m, 1.0, 0.0)
                    oxr = slab_x[i, pl.ds(js + 16 * j, 16)]
                    ax = ax + jnp.where(m, jnp.abs(oxr - rampx), 0.0)
                    oyr = slab_y[i, pl.ds(js + 16 * j, 16)]
                    rampy = dc.astype(jnp.float32) * (-1.0 / _RADIUS)
                    ay = ay + jnp.where(m, jnp.abs(oyr - rampy), 0.0)
                return (d, n, ax, ay)

            d, n, ax, ay = lax.fori_loop(
                0, _ROWS, row_body, (zero, zero, zero, zero))
            accv[0, :] = d
            accv[1, :] = n
            accv[2, :] = ax
            accv[3, :] = ay
            pltpu.async_copy(accv, o_hbm.at[g], sem_o).wait()


def _window_sums(feature_maps, xy):
    B, F, h, w = feature_maps.shape
    P = F // 3
    G = B * P
    mesh = plsc.VectorSubcoreMesh(core_axis_name="c", subcore_axis_name="s")
    body = pl.kernel(
        functools.partial(_window_body, B, P, h, w),
        out_type=jax.ShapeDtypeStruct((G, 4, _LANES), jnp.float32),
        mesh=mesh,
        scratch_types=[
            pltpu.VMEM((_ROWS, _SLABC), jnp.float32),
            pltpu.VMEM((_ROWS, _SLABC), jnp.float32),
            pltpu.VMEM((_ROWS, _SLABC), jnp.float32),
            pltpu.VMEM((4, _LANES), jnp.float32),
            pltpu.VMEM((((2 * G + 16 + 15) // 16) * 16,), jnp.int32),
            pltpu.SemaphoreType.DMA,
            pltpu.SemaphoreType.DMA,
            pltpu.SemaphoreType.DMA,
            pltpu.SemaphoreType.DMA,
        ],
    )
    return body(feature_maps, xy)


def kernel(feature_maps, landmarks):
    B, F, h, w = feature_maps.shape
    P = F // 3
    G = B * P
    lm = (landmarks * jnp.array([h, w], dtype=jnp.float32)).astype(jnp.int32)
    xy = lm.reshape(-1)
    pad = ((2 * G + 16 + 15) // 16) * 16 - 2 * G
    xy = jnp.pad(xy, (0, pad))

    S = _dense_softplus_sum(feature_maps)
    wsums = jnp.sum(_window_sums(feature_maps, xy), axis=-1)
    disk = wsums[:, 0]
    cnt = wsums[:, 1]
    l1 = wsums[:, 2] + wsums[:, 3]
    return 2.0 * (S - jnp.sum(disk)) / (G * h * w) + jnp.mean(l1 / cnt)


# speedup vs baseline: 10.2391x; 1.0167x over previous
import functools

import jax
import jax.numpy as jnp
from jax import lax
from jax.experimental import pallas as pl
from jax.experimental.pallas import tpu as pltpu
from jax.experimental.pallas import tpu_sc as plsc

_RADIUS = 40
_RAD2 = _RADIUS * _RADIUS
_ROWS = 88
_SLABC = 256
_COLS = 96
_LANES = 16


def _dense_body(P, l_ref, out_ref, acc_ref):
    @pl.when((pl.program_id(0) == 0) & (pl.program_id(1) == 0))
    def _():
        acc_ref[...] = jnp.zeros(acc_ref.shape, acc_ref.dtype)

    acc = acc_ref[...]
    for ch in range(P):
        t = l_ref[0, ch]
        e = jnp.exp2(jnp.abs(t) * (-1.4426950408889634))
        acc = acc + (jnp.maximum(t, 0.0)
                     + jnp.log2(1.0 + e) * 0.6931471805599453)
    acc_ref[...] = acc

    @pl.when((pl.program_id(0) == pl.num_programs(0) - 1)
             & (pl.program_id(1) == pl.num_programs(1) - 1))
    def _():
        a = acc_ref[...]
        a8 = a[0:8] + a[8:16] + a[16:24] + a[24:32]
        out_ref[...] = (a8[:, 0:128] + a8[:, 128:256]
                        + a8[:, 256:384] + a8[:, 384:512])


def _dense_softplus_sum(feature_maps):
    B, F, h, w = feature_maps.shape
    P = F // 3
    out = pl.pallas_call(
        functools.partial(_dense_body, P),
        grid=(B, h // 32),
        in_specs=[pl.BlockSpec((1, P, 32, w), lambda b, r: (b, 0, r, 0))],
        out_specs=pl.BlockSpec((8, 128), lambda b, r: (0, 0)),
        out_shape=jax.ShapeDtypeStruct((8, 128), jnp.float32),
        scratch_shapes=[pltpu.VMEM((32, w), jnp.float32)],
    )(feature_maps)
    return jnp.sum(out)


def _window_body(B, P, h, w, fm, xy_hbm, o_hbm,
                 slab_l, slab_x, slab_y, accv, xy_smem,
                 sem_l, sem_x, sem_y, sem_l2, sem_x2, sem_y2, sem_o):
    G = B * P
    cid = lax.axis_index("c")
    sid = lax.axis_index("s")
    idx = cid * 16 + sid
    pltpu.async_copy(xy_hbm, xy_smem, sem_l).wait()
    lane = lax.broadcasted_iota(jnp.int32, (_LANES,), 0)
    scale = jnp.where((lane & 1) == 0, jnp.float32(h), jnp.float32(w))

    for k in range((G + 31) // 32):
        g = idx + 32 * k

        @pl.when(g < G)
        def _():
            b = g // P
            p = g % P
            xyv = (xy_smem[pl.ds(2 * g, _LANES)] * scale).astype(jnp.int32)
            x = xyv[0]
            y = xyv[1]
            rs = jnp.minimum((jnp.maximum(x - _RADIUS, 0) // 8) * 8, h - _ROWS)
            cs = jnp.minimum((jnp.maximum(y - _RADIUS, 0) // 128) * 128,
                             w - 128)
            rs = pl.multiple_of(rs, 8)
            cs = pl.multiple_of(cs, 128)
            cs2 = pl.multiple_of(cs + 128, 128)
            need2 = (jnp.minimum(y + _RADIUS, w - 1) // 128) * 128 > cs
            js = ((jnp.maximum(y - _RADIUS, 0) - cs) // _LANES) * _LANES
            cp_l = pltpu.async_copy(
                fm.at[b, p, pl.ds(rs, _ROWS), pl.ds(cs, 128)],
                slab_l.at[:, 0:128], sem_l)
            cp_x = pltpu.async_copy(
                fm.at[b, P + p, pl.ds(rs, _ROWS), pl.ds(cs, 128)],
                slab_x.at[:, 0:128], sem_x)
            cp_y = pltpu.async_copy(
                fm.at[b, 2 * P + p, pl.ds(rs, _ROWS), pl.ds(cs, 128)],
                slab_y.at[:, 0:128], sem_y)

            @pl.when(need2)
            def _():
                cp2_l = pltpu.async_copy(
                    fm.at[b, p, pl.ds(rs, _ROWS), pl.ds(cs2, 128)],
                    slab_l.at[:, 128:256], sem_l2)
                cp2_x = pltpu.async_copy(
                    fm.at[b, P + p, pl.ds(rs, _ROWS), pl.ds(cs2, 128)],
                    slab_x.at[:, 128:256], sem_x2)
                cp2_y = pltpu.async_copy(
                    fm.at[b, 2 * P + p, pl.ds(rs, _ROWS), pl.ds(cs2, 128)],
                    slab_y.at[:, 128:256], sem_y2)
                cp2_l.wait()
                cp2_x.wait()
                cp2_y.wait()

            cp_l.wait()
            cp_x.wait()
            cp_y.wait()

            zero = jnp.zeros((_LANES,), jnp.float32)

            def row_body(i, accs):
                d, n, ax, ay = accs
                r = rs + i
                dr = r - x
                dr2 = dr * dr
                rampx = (x - r).astype(jnp.float32) * (1.0 / _RADIUS)
                for j in range(_COLS // _LANES):
                    dc = cs + js + (16 * j) + lane - y
                    m = (dc * dc + dr2) <= _RAD2
                    lrow = slab_l[i, pl.ds(js + 16 * j, 16)]
                    d = d + jnp.where(m, lrow, 0.0)
                    n = n + jnp.where(m, 1.0, 0.0)
                    oxr = slab_x[i, pl.ds(js + 16 * j, 16)]
                    ax = ax + jnp.where(m, jnp.abs(oxr - rampx), 0.0)
                    oyr = slab_y[i, pl.ds(js + 16 * j, 16)]
                    rampy = dc.astype(jnp.float32) * (-1.0 / _RADIUS)
                    ay = ay + jnp.where(m, jnp.abs(oyr - rampy), 0.0)
                return (d, n, ax, ay)

            d, n, ax, ay = lax.fori_loop(
                0, _ROWS, row_body, (zero, zero, zero, zero))
            accv[0, :] = d
            accv[1, :] = n
            accv[2, :] = ax
            accv[3, :] = ay
            pltpu.async_copy(accv, o_hbm.at[g], sem_o).wait()


def _window_sums(feature_maps, xy):
    B, F, h, w = feature_maps.shape
    P = F // 3
    G = B * P
    mesh = plsc.VectorSubcoreMesh(core_axis_name="c", subcore_axis_name="s")
    body = pl.kernel(
        functools.partial(_window_body, B, P, h, w),
        out_type=jax.ShapeDtypeStruct((G, 4, _LANES), jnp.float32),
        mesh=mesh,
        scratch_types=[
            pltpu.VMEM((_ROWS, _SLABC), jnp.float32),
            pltpu.VMEM((_ROWS, _SLABC), jnp.float32),
            pltpu.VMEM((_ROWS, _SLABC), jnp.float32),
            pltpu.VMEM((4, _LANES), jnp.float32),
            pltpu.VMEM((((2 * G + 16 + 15) // 16) * 16,), jnp.float32),
            pltpu.SemaphoreType.DMA,
            pltpu.SemaphoreType.DMA,
            pltpu.SemaphoreType.DMA,
            pltpu.SemaphoreType.DMA,
            pltpu.SemaphoreType.DMA,
            pltpu.SemaphoreType.DMA,
            pltpu.SemaphoreType.DMA,
        ],
    )
    return body(feature_maps, xy)


def kernel(feature_maps, landmarks):
    B, F, h, w = feature_maps.shape
    P = F // 3
    G = B * P
    xy = landmarks.reshape(-1)
    pad = ((2 * G + 16 + 15) // 16) * 16 - 2 * G
    xy = jnp.pad(xy, (0, pad))

    S = _dense_softplus_sum(feature_maps)
    wsums = jnp.sum(_window_sums(feature_maps, xy), axis=-1)
    disk = wsums[:, 0]
    cnt = wsums[:, 1]
    l1 = wsums[:, 2] + wsums[:, 3]
    return 2.0 * (S - jnp.sum(disk)) / (G * h * w) + jnp.mean(l1 / cnt)


# speedup vs baseline: 10.5762x; 1.0329x over previous
import functools

import jax
import jax.numpy as jnp
from jax import lax
from jax.experimental import pallas as pl
from jax.experimental.pallas import tpu as pltpu
from jax.experimental.pallas import tpu_sc as plsc

_RADIUS = 40
_RAD2 = _RADIUS * _RADIUS
_ROWS = 88
_SLABC = 256
_COLS = 96
_LANES = 16


def _dense_body(P, l_ref, out_ref, acc_ref):
    @pl.when((pl.program_id(0) == 0) & (pl.program_id(1) == 0))
    def _():
        acc_ref[...] = jnp.zeros(acc_ref.shape, acc_ref.dtype)

    acc = acc_ref[...]
    for ch in range(P):
        t = l_ref[0, ch]
        e = jnp.exp2(jnp.abs(t) * (-1.4426950408889634))
        u = 1.0 + e
        pr = (u[0:8] * u[8:16]) * (u[16:24] * u[24:32])
        mx = ((jnp.maximum(t[0:8], 0.0) + jnp.maximum(t[8:16], 0.0))
              + (jnp.maximum(t[16:24], 0.0) + jnp.maximum(t[24:32], 0.0)))
        acc = acc + (mx + jnp.log2(pr) * 0.6931471805599453)
    acc_ref[...] = acc

    @pl.when((pl.program_id(0) == pl.num_programs(0) - 1)
             & (pl.program_id(1) == pl.num_programs(1) - 1))
    def _():
        a8 = acc_ref[...]
        out_ref[...] = (a8[:, 0:128] + a8[:, 128:256]
                        + a8[:, 256:384] + a8[:, 384:512])


def _dense_softplus_sum(feature_maps):
    B, F, h, w = feature_maps.shape
    P = F // 3
    out = pl.pallas_call(
        functools.partial(_dense_body, P),
        grid=(B, h // 32),
        in_specs=[pl.BlockSpec((1, P, 32, w), lambda b, r: (b, 0, r, 0))],
        out_specs=pl.BlockSpec((8, 128), lambda b, r: (0, 0)),
        out_shape=jax.ShapeDtypeStruct((8, 128), jnp.float32),
        scratch_shapes=[pltpu.VMEM((8, w), jnp.float32)],
    )(feature_maps)
    return jnp.sum(out)


def _window_body(B, P, h, w, fm, xy_hbm, o_hbm,
                 slab_l, slab_x, slab_y, accv, xy_smem,
                 sem_l, sem_x, sem_y, sem_l2, sem_x2, sem_y2, sem_o):
    G = B * P
    cid = lax.axis_index("c")
    sid = lax.axis_index("s")
    idx = cid * 16 + sid
    pltpu.async_copy(xy_hbm, xy_smem, sem_l).wait()
    lane = lax.broadcasted_iota(jnp.int32, (_LANES,), 0)
    scale = jnp.where((lane & 1) == 0, jnp.float32(h), jnp.float32(w))

    for k in range((G + 31) // 32):
        g = idx + 32 * k

        @pl.when(g < G)
        def _():
            b = g // P
            p = g % P
            xyv = (xy_smem[pl.ds(2 * g, _LANES)] * scale).astype(jnp.int32)
            x = xyv[0]
            y = xyv[1]
            rs = jnp.minimum((jnp.maximum(x - _RADIUS, 0) // 8) * 8, h - _ROWS)
            cs = jnp.minimum((jnp.maximum(y - _RADIUS, 0) // 128) * 128,
                             w - 128)
            rs = pl.multiple_of(rs, 8)
            cs = pl.multiple_of(cs, 128)
            cs2 = pl.multiple_of(cs + 128, 128)
            need2 = (jnp.minimum(y + _RADIUS, w - 1) // 128) * 128 > cs
            js = ((jnp.maximum(y - _RADIUS, 0) - cs) // _LANES) * _LANES
            cp_l = pltpu.async_copy(
                fm.at[b, p, pl.ds(rs, _ROWS), pl.ds(cs, 128)],
                slab_l.at[:, 0:128], sem_l)
            cp_x = pltpu.async_copy(
                fm.at[b, P + p, pl.ds(rs, _ROWS), pl.ds(cs, 128)],
                slab_x.at[:, 0:128], sem_x)
            cp_y = pltpu.async_copy(
                fm.at[b, 2 * P + p, pl.ds(rs, _ROWS), pl.ds(cs, 128)],
                slab_y.at[:, 0:128], sem_y)

            @pl.when(need2)
            def _():
                cp2_l = pltpu.async_copy(
                    fm.at[b, p, pl.ds(rs, _ROWS), pl.ds(cs2, 128)],
                    slab_l.at[:, 128:256], sem_l2)
                cp2_x = pltpu.async_copy(
                    fm.at[b, P + p, pl.ds(rs, _ROWS), pl.ds(cs2, 128)],
                    slab_x.at[:, 128:256], sem_x2)
                cp2_y = pltpu.async_copy(
                    fm.at[b, 2 * P + p, pl.ds(rs, _ROWS), pl.ds(cs2, 128)],
                    slab_y.at[:, 128:256], sem_y2)
                cp2_l.wait()
                cp2_x.wait()
                cp2_y.wait()

            cp_l.wait()
            cp_x.wait()
            cp_y.wait()

            zero = jnp.zeros((_LANES,), jnp.float32)

            def row_body(i, accs):
                d, n, ax, ay = accs
                r = rs + i
                dr = r - x
                dr2 = dr * dr
                rampx = (x - r).astype(jnp.float32) * (1.0 / _RADIUS)
                for j in range(_COLS // _LANES):
                    dc = cs + js + (16 * j) + lane - y
                    m = (dc * dc + dr2) <= _RAD2
                    lrow = slab_l[i, pl.ds(js + 16 * j, 16)]
                    d = d + jnp.where(m, lrow, 0.0)
                    n = n + jnp.where(m, 1.0, 0.0)
                    oxr = slab_x[i, pl.ds(js + 16 * j, 16)]
                    ax = ax + jnp.where(m, jnp.abs(oxr - rampx), 0.0)
                    oyr = slab_y[i, pl.ds(js + 16 * j, 16)]
                    rampy = dc.astype(jnp.float32) * (-1.0 / _RADIUS)
                    ay = ay + jnp.where(m, jnp.abs(oyr - rampy), 0.0)
                return (d, n, ax, ay)

            d, n, ax, ay = lax.fori_loop(
                0, _ROWS, row_body, (zero, zero, zero, zero))
            accv[0, :] = d
            accv[1, :] = n
            accv[2, :] = ax
            accv[3, :] = ay
            pltpu.async_copy(accv, o_hbm.at[g], sem_o).wait()


def _window_sums(feature_maps, xy):
    B, F, h, w = feature_maps.shape
    P = F // 3
    G = B * P
    mesh = plsc.VectorSubcoreMesh(core_axis_name="c", subcore_axis_name="s")
    body = pl.kernel(
        functools.partial(_window_body, B, P, h, w),
        out_type=jax.ShapeDtypeStruct((G, 4, _LANES), jnp.float32),
        mesh=mesh,
        scratch_types=[
            pltpu.VMEM((_ROWS, _SLABC), jnp.float32),
            pltpu.VMEM((_ROWS, _SLABC), jnp.float32),
            pltpu.VMEM((_ROWS, _SLABC), jnp.float32),
            pltpu.VMEM((4, _LANES), jnp.float32),
            pltpu.VMEM((((2 * G + 16 + 15) // 16) * 16,), jnp.float32),
            pltpu.SemaphoreType.DMA,
            pltpu.SemaphoreType.DMA,
            pltpu.SemaphoreType.DMA,
            pltpu.SemaphoreType.DMA,
            pltpu.SemaphoreType.DMA,
            pltpu.SemaphoreType.DMA,
            pltpu.SemaphoreType.DMA,
        ],
    )
    return body(feature_maps, xy)


def kernel(feature_maps, landmarks):
    B, F, h, w = feature_maps.shape
    P = F // 3
    G = B * P
    xy = landmarks.reshape(-1)
    pad = ((2 * G + 16 + 15) // 16) * 16 - 2 * G
    xy = jnp.pad(xy, (0, pad))

    S = _dense_softplus_sum(feature_maps)
    wsums = jnp.sum(_window_sums(feature_maps, xy), axis=-1)
    disk = wsums[:, 0]
    cnt = wsums[:, 1]
    l1 = wsums[:, 2] + wsums[:, 3]
    return 2.0 * (S - jnp.sum(disk)) / (G * h * w) + jnp.mean(l1 / cnt)


# speedup vs baseline: 10.6084x; 1.0030x over previous
import functools

import jax
import jax.numpy as jnp
from jax import lax
from jax.experimental import pallas as pl
from jax.experimental.pallas import tpu as pltpu
from jax.experimental.pallas import tpu_sc as plsc

_RADIUS = 40
_RAD2 = _RADIUS * _RADIUS
_ROWS = 88
_SLABC = 256
_COLS = 96
_LANES = 16


def _dense_body(P, l_ref, out_ref, acc_ref):
    @pl.when((pl.program_id(0) == 0) & (pl.program_id(1) == 0))
    def _():
        acc_ref[...] = jnp.zeros(acc_ref.shape, acc_ref.dtype)

    acc = acc_ref[...]
    for ch in range(P):
        t = l_ref[0, ch]
        e = jnp.exp2(jnp.abs(t) * (-1.4426950408889634))
        u = 1.0 + e
        pr = (u[0:8] * u[8:16]) * (u[16:24] * u[24:32])
        mx = ((jnp.maximum(t[0:8], 0.0) + jnp.maximum(t[8:16], 0.0))
              + (jnp.maximum(t[16:24], 0.0) + jnp.maximum(t[24:32], 0.0)))
        acc = acc + (mx + jnp.log2(pr) * 0.6931471805599453)
    acc_ref[...] = acc

    @pl.when((pl.program_id(0) == pl.num_programs(0) - 1)
             & (pl.program_id(1) == pl.num_programs(1) - 1))
    def _():
        a8 = acc_ref[...]
        out_ref[...] = (a8[:, 0:128] + a8[:, 128:256]
                        + a8[:, 256:384] + a8[:, 384:512])


def _dense_softplus_sum(feature_maps):
    B, F, h, w = feature_maps.shape
    P = F // 3
    out = pl.pallas_call(
        functools.partial(_dense_body, P),
        grid=(B, h // 32),
        in_specs=[pl.BlockSpec((1, P, 32, w), lambda b, r: (b, 0, r, 0))],
        out_specs=pl.BlockSpec((8, 128), lambda b, r: (0, 0)),
        out_shape=jax.ShapeDtypeStruct((8, 128), jnp.float32),
        scratch_shapes=[pltpu.VMEM((8, w), jnp.float32)],
    )(feature_maps)
    return jnp.sum(out)


def _window_body(B, P, h, w, fm, xy_hbm, o_hbm,
                 slab_l, slab_x, slab_y, accv, xy_smem,
                 sem_l, sem_x, sem_y, sem_l2, sem_x2, sem_y2, sem_o):
    G = B * P
    cid = lax.axis_index("c")
    sid = lax.axis_index("s")
    idx = cid * 16 + sid
    pltpu.async_copy(xy_hbm, xy_smem, sem_l).wait()
    lane = lax.broadcasted_iota(jnp.int32, (_LANES,), 0)
    scale = jnp.where((lane & 1) == 0, jnp.float32(h), jnp.float32(w))

    @pl.loop(0, (G + 31) // 32)
    def _(k):
        g = idx + 32 * k

        @pl.when(g < G)
        def _():
            b = g // P
            p = g % P
            xyv = (xy_smem[pl.ds(2 * g, _LANES)] * scale).astype(jnp.int32)
            x = xyv[0]
            y = xyv[1]
            rs = jnp.minimum((jnp.maximum(x - _RADIUS, 0) // 8) * 8, h - _ROWS)
            cs = jnp.minimum((jnp.maximum(y - _RADIUS, 0) // 128) * 128,
                             w - 128)
            rs = pl.multiple_of(rs, 8)
            cs = pl.multiple_of(cs, 128)
            cs2 = pl.multiple_of(cs + 128, 128)
            need2 = (jnp.minimum(y + _RADIUS, w - 1) // 128) * 128 > cs
            js = ((jnp.maximum(y - _RADIUS, 0) - cs) // _LANES) * _LANES
            cp_l = pltpu.async_copy(
                fm.at[b, p, pl.ds(rs, _ROWS), pl.ds(cs, 128)],
                slab_l.at[:, 0:128], sem_l)
            cp_x = pltpu.async_copy(
                fm.at[b, P + p, pl.ds(rs, _ROWS), pl.ds(cs, 128)],
                slab_x.at[:, 0:128], sem_x)
            cp_y = pltpu.async_copy(
                fm.at[b, 2 * P + p, pl.ds(rs, _ROWS), pl.ds(cs, 128)],
                slab_y.at[:, 0:128], sem_y)

            @pl.when(need2)
            def _():
                cp2_l = pltpu.async_copy(
                    fm.at[b, p, pl.ds(rs, _ROWS), pl.ds(cs2, 128)],
                    slab_l.at[:, 128:256], sem_l2)
                cp2_x = pltpu.async_copy(
                    fm.at[b, P + p, pl.ds(rs, _ROWS), pl.ds(cs2, 128)],
                    slab_x.at[:, 128:256], sem_x2)
                cp2_y = pltpu.async_copy(
                    fm.at[b, 2 * P + p, pl.ds(rs, _ROWS), pl.ds(cs2, 128)],
                    slab_y.at[:, 128:256], sem_y2)
                cp2_l.wait()
                cp2_x.wait()
                cp2_y.wait()

            cp_l.wait()
            cp_x.wait()
            cp_y.wait()

            zero = jnp.zeros((_LANES,), jnp.float32)

            def row_body(i, accs):
                d, n, ax, ay = accs
                r = rs + i
                dr = r - x
                dr2 = dr * dr
                rampx = (x - r).astype(jnp.float32) * (1.0 / _RADIUS)
                for j in range(_COLS // _LANES):
                    dc = cs + js + (16 * j) + lane - y
                    m = (dc * dc + dr2) <= _RAD2
                    lrow = slab_l[i, pl.ds(js + 16 * j, 16)]
                    d = d + jnp.where(m, lrow, 0.0)
                    n = n + jnp.where(m, 1.0, 0.0)
                    oxr = slab_x[i, pl.ds(js + 16 * j, 16)]
                    ax = ax + jnp.where(m, jnp.abs(oxr - rampx), 0.0)
                    oyr = slab_y[i, pl.ds(js + 16 * j, 16)]
                    rampy = dc.astype(jnp.float32) * (-1.0 / _RADIUS)
                    ay = ay + jnp.where(m, jnp.abs(oyr - rampy), 0.0)
                return (d, n, ax, ay)

            d, n, ax, ay = lax.fori_loop(
                0, _ROWS, row_body, (zero, zero, zero, zero))
            accv[0, :] = d
            accv[1, :] = n
            accv[2, :] = ax
            accv[3, :] = ay
            pltpu.async_copy(accv, o_hbm.at[g], sem_o).wait()


def _window_sums(feature_maps, xy):
    B, F, h, w = feature_maps.shape
    P = F // 3
    G = B * P
    mesh = plsc.VectorSubcoreMesh(core_axis_name="c", subcore_axis_name="s")
    body = pl.kernel(
        functools.partial(_window_body, B, P, h, w),
        out_type=jax.ShapeDtypeStruct((G, 4, _LANES), jnp.float32),
        mesh=mesh,
        scratch_types=[
            pltpu.VMEM((_ROWS, _SLABC), jnp.float32),
            pltpu.VMEM((_ROWS, _SLABC), jnp.float32),
            pltpu.VMEM((_ROWS, _SLABC), jnp.float32),
            pltpu.VMEM((4, _LANES), jnp.float32),
            pltpu.VMEM((((2 * G + 16 + 15) // 16) * 16,), jnp.float32),
            pltpu.SemaphoreType.DMA,
            pltpu.SemaphoreType.DMA,
            pltpu.SemaphoreType.DMA,
            pltpu.SemaphoreType.DMA,
            pltpu.SemaphoreType.DMA,
            pltpu.SemaphoreType.DMA,
            pltpu.SemaphoreType.DMA,
        ],
    )
    return body(feature_maps, xy)


def kernel(feature_maps, landmarks):
    B, F, h, w = feature_maps.shape
    P = F // 3
    G = B * P
    xy = landmarks.reshape(-1)
    pad = ((2 * G + 16 + 15) // 16) * 16 - 2 * G
    xy = jnp.pad(xy, (0, pad))

    S = _dense_softplus_sum(feature_maps)
    wsums = jnp.sum(_window_sums(feature_maps, xy), axis=-1)
    disk = wsums[:, 0]
    cnt = wsums[:, 1]
    l1 = wsums[:, 2] + wsums[:, 3]
    return 2.0 * (S - jnp.sum(disk)) / (G * h * w) + jnp.mean(l1 / cnt)


# speedup vs baseline: 10.6801x; 1.0068x over previous
import dataclasses
import functools

import jax
import jax.numpy as jnp
from jax import lax
from jax.experimental import pallas as pl
from jax.experimental.pallas import tpu as pltpu
from jax.experimental.pallas import tpu_sc as plsc

_RADIUS = 40
_RAD2 = _RADIUS * _RADIUS
_ROWS = 88
_SLABC = 256
_COLS = 96
_LANES = 16


def _dense_body(P, l_ref, out_ref, acc_ref):
    @pl.when((pl.program_id(0) == 0) & (pl.program_id(1) == 0))
    def _():
        acc_ref[...] = jnp.zeros(acc_ref.shape, acc_ref.dtype)

    acc = acc_ref[...]
    for ch in range(P):
        t = l_ref[0, ch]
        e = jnp.exp2(jnp.abs(t) * (-1.4426950408889634))
        u = 1.0 + e
        pr = (u[0:8] * u[8:16]) * (u[16:24] * u[24:32])
        mx = ((jnp.maximum(t[0:8], 0.0) + jnp.maximum(t[8:16], 0.0))
              + (jnp.maximum(t[16:24], 0.0) + jnp.maximum(t[24:32], 0.0)))
        acc = acc + (mx + jnp.log2(pr) * 0.6931471805599453)
    acc_ref[...] = acc

    @pl.when((pl.program_id(0) == pl.num_programs(0) - 1)
             & (pl.program_id(1) == pl.num_programs(1) - 1))
    def _():
        out_ref[0, 0] = jnp.sum(acc_ref[...])


def _dense_softplus_sum(feature_maps):
    B, F, h, w = feature_maps.shape
    P = F // 3
    out = pl.pallas_call(
        functools.partial(_dense_body, P),
        grid=(B, h // 32),
        in_specs=[pl.BlockSpec((1, P, 32, w), lambda b, r: (b, 0, r, 0))],
        out_specs=pl.BlockSpec(memory_space=pltpu.SMEM),
        out_shape=jax.ShapeDtypeStruct((1, 1), jnp.float32),
        scratch_shapes=[pltpu.VMEM((8, w), jnp.float32)],
    )(feature_maps)
    return out[0, 0]


def _window_body(B, P, h, w, fm, xy_hbm, o_hbm,
                 slab_l, slab_x, slab_y, accv, xy_smem,
                 sem_l, sem_x, sem_y, sem_l2, sem_x2, sem_y2, sem_o):
    G = B * P
    cid = lax.axis_index("c")
    sid = lax.axis_index("s")
    idx = cid * 16 + sid
    pltpu.async_copy(xy_hbm, xy_smem, sem_l).wait()
    lane = lax.broadcasted_iota(jnp.int32, (_LANES,), 0)
    scale = jnp.where((lane & 1) == 0, jnp.float32(h), jnp.float32(w))

    @pl.loop(0, (G + 31) // 32)
    def _(k):
        g = idx + 32 * k

        @pl.when(g < G)
        def _():
            b = g // P
            p = g % P
            xyv = (xy_smem[pl.ds(2 * g, _LANES)] * scale).astype(jnp.int32)
            x = xyv[0]
            y = xyv[1]
            rs = jnp.minimum((jnp.maximum(x - _RADIUS, 0) // 8) * 8, h - _ROWS)
            cs = jnp.minimum((jnp.maximum(y - _RADIUS, 0) // 128) * 128,
                             w - 128)
            rs = pl.multiple_of(rs, 8)
            cs = pl.multiple_of(cs, 128)
            cs2 = pl.multiple_of(cs + 128, 128)
            need2 = (jnp.minimum(y + _RADIUS, w - 1) // 128) * 128 > cs
            js = ((jnp.maximum(y - _RADIUS, 0) - cs) // _LANES) * _LANES
            cp_l = pltpu.async_copy(
                fm.at[b, p, pl.ds(rs, _ROWS), pl.ds(cs, 128)],
                slab_l.at[:, 0:128], sem_l)
            cp_x = pltpu.async_copy(
                fm.at[b, P + p, pl.ds(rs, _ROWS), pl.ds(cs, 128)],
                slab_x.at[:, 0:128], sem_x)
            cp_y = pltpu.async_copy(
                fm.at[b, 2 * P + p, pl.ds(rs, _ROWS), pl.ds(cs, 128)],
                slab_y.at[:, 0:128], sem_y)

            @pl.when(need2)
            def _():
                cp2_l = pltpu.async_copy(
                    fm.at[b, p, pl.ds(rs, _ROWS), pl.ds(cs2, 128)],
                    slab_l.at[:, 128:256], sem_l2)
                cp2_x = pltpu.async_copy(
                    fm.at[b, P + p, pl.ds(rs, _ROWS), pl.ds(cs2, 128)],
                    slab_x.at[:, 128:256], sem_x2)
                cp2_y = pltpu.async_copy(
                    fm.at[b, 2 * P + p, pl.ds(rs, _ROWS), pl.ds(cs2, 128)],
                    slab_y.at[:, 128:256], sem_y2)
                cp2_l.wait()
                cp2_x.wait()
                cp2_y.wait()

            cp_l.wait()
            cp_x.wait()
            cp_y.wait()

            zero = jnp.zeros((_LANES,), jnp.float32)

            def row_body(i, accs):
                d, n, ax, ay = accs
                r = rs + i
                dr = r - x
                dr2 = dr * dr
                rampx = (x - r).astype(jnp.float32) * (1.0 / _RADIUS)
                for j in range(_COLS // _LANES):
                    dc = cs + js + (16 * j) + lane - y
                    m = (dc * dc + dr2) <= _RAD2
                    lrow = slab_l[i, pl.ds(js + 16 * j, 16)]
                    d = d + jnp.where(m, lrow, 0.0)
                    n = n + jnp.where(m, 1.0, 0.0)
                    oxr = slab_x[i, pl.ds(js + 16 * j, 16)]
                    ax = ax + jnp.where(m, jnp.abs(oxr - rampx), 0.0)
                    oyr = slab_y[i, pl.ds(js + 16 * j, 16)]
                    rampy = dc.astype(jnp.float32) * (-1.0 / _RADIUS)
                    ay = ay + jnp.where(m, jnp.abs(oyr - rampy), 0.0)
                return (d, n, ax, ay)

            d, n, ax, ay = lax.fori_loop(
                0, _ROWS, row_body, (zero, zero, zero, zero))
            accv[0, :] = jnp.full((_LANES,), jnp.sum(d), jnp.float32)
            accv[1, :] = jnp.full((_LANES,), jnp.sum(n), jnp.float32)
            accv[2, :] = jnp.full((_LANES,), jnp.sum(ax), jnp.float32)
            accv[3, :] = jnp.full((_LANES,), jnp.sum(ay), jnp.float32)
            pltpu.async_copy(accv, o_hbm.at[g], sem_o).wait()


def _window_sums(feature_maps, xy):
    B, F, h, w = feature_maps.shape
    P = F // 3
    G = B * P
    mesh = plsc.VectorSubcoreMesh(core_axis_name="c", subcore_axis_name="s")
    cp = pltpu.CompilerParams()
    if "needs_layout_passes" in pltpu.CompilerParams.__dataclass_fields__:
        cp = dataclasses.replace(cp, needs_layout_passes=False)
    body = pl.kernel(
        functools.partial(_window_body, B, P, h, w),
        out_type=jax.ShapeDtypeStruct((G, 4, _LANES), jnp.float32),
        mesh=mesh,
        compiler_params=cp,
        scratch_types=[
            pltpu.VMEM((_ROWS, _SLABC), jnp.float32),
            pltpu.VMEM((_ROWS, _SLABC), jnp.float32),
            pltpu.VMEM((_ROWS, _SLABC), jnp.float32),
            pltpu.VMEM((4, _LANES), jnp.float32),
            pltpu.VMEM((((2 * G + 16 + 15) // 16) * 16,), jnp.float32),
            pltpu.SemaphoreType.DMA,
            pltpu.SemaphoreType.DMA,
            pltpu.SemaphoreType.DMA,
            pltpu.SemaphoreType.DMA,
            pltpu.SemaphoreType.DMA,
            pltpu.SemaphoreType.DMA,
            pltpu.SemaphoreType.DMA,
        ],
    )
    return body(feature_maps, xy)


def kernel(feature_maps, landmarks):
    B, F, h, w = feature_maps.shape
    P = F // 3
    G = B * P
    xy = landmarks.reshape(-1)
    pad = ((2 * G + 16 + 15) // 16) * 16 - 2 * G
    xy = jnp.pad(xy, (0, pad))

    S = _dense_softplus_sum(feature_maps)
    wsums = _window_sums(feature_maps, xy)[:, :, 0]
    disk = wsums[:, 0]
    cnt = wsums[:, 1]
    l1 = wsums[:, 2] + wsums[:, 3]
    return 2.0 * (S - jnp.sum(disk)) / (G * h * w) + jnp.mean(l1 / cnt)
